# baseline (speedup 1.0000x reference)
import functools

import jax
import jax.numpy as jnp
from jax import lax
from jax.experimental import pallas as pl
from jax.experimental.pallas import tpu as pltpu
from jax.experimental.pallas import tpu_sc as plsc

_PT = (0.6, 0.7, 0.8, 0.9)
_B = 16384
_H = 5
_D = 1024
_DO = 512
_T = 256
_NTOT = _B + _H * _T
_NT = _NTOT // _T
_NHPAD = 80
_NC = 2
_NS = 16
_NW = _NC * _NS
_R = _B // _NW
_NG = _R // 16
_NBUF = 4

_mesh = plsc.VectorSubcoreMesh(core_axis_name="c", subcore_axis_name="s")
_sc_params = pltpu.CompilerParams(needs_layout_passes=False)


@functools.partial(
    pl.kernel, mesh=_mesh, compiler_params=_sc_params,
    out_type=[jax.ShapeDtypeStruct((_B,), jnp.int32),
              jax.ShapeDtypeStruct((_NW * 16,), jnp.int32)],
    scratch_types=[pltpu.VMEM((_R,), jnp.float32),
                   pltpu.VMEM((_R,), jnp.int32),
                   pltpu.VMEM((16,), jnp.int32)],
)
def _sc_bin_hist(t_hbm, bins_hbm, hist_hbm, t_v, bins_v, hist_v):
    wid = lax.axis_index("s") * _NC + lax.axis_index("c")
    base = wid * _R
    lane = lax.iota(jnp.int32, 16)
    pltpu.sync_copy(t_hbm.at[pl.ds(base, _R)], t_v)

    def body(g, hist):
        tv = t_v[pl.ds(g * 16, 16)]
        bv = sum((tv >= p).astype(jnp.int32) for p in _PT)
        bins_v[pl.ds(g * 16, 16)] = bv
        for h in range(_H):
            cnt = jnp.sum((bv == h).astype(jnp.int32))
            hist = hist + jnp.where(lane == h, cnt, 0)
        return hist

    hist = lax.fori_loop(0, _NG, body, jnp.zeros((16,), jnp.int32))
    hist_v[...] = hist
    pltpu.sync_copy(bins_v, bins_hbm.at[pl.ds(base, _R)])
    pltpu.sync_copy(hist_v, hist_hbm.at[pl.ds(wid * 16, 16)])


@functools.partial(
    pl.kernel, mesh=_mesh, compiler_params=_sc_params,
    out_type=[jax.ShapeDtypeStruct((_NTOT, _D), jnp.float32),
              jax.ShapeDtypeStruct((_NTOT,), jnp.float32),
              jax.ShapeDtypeStruct((_B,), jnp.int32),
              jax.ShapeDtypeStruct((_NHPAD,), jnp.int32)],
    scratch_types=[pltpu.VMEM((_NW * 16,), jnp.int32),
                   pltpu.VMEM((_R,), jnp.int32),
                   pltpu.VMEM((_R,), jnp.int32),
                   pltpu.VMEM((_R,), jnp.float32),
                   pltpu.VMEM((_NBUF, 16, _D), jnp.float32),
                   pltpu.VMEM((_NHPAD,), jnp.int32),
                   pltpu.SemaphoreType.DMA,
                   pltpu.SemaphoreType.DMA,
                   pltpu.SemaphoreType.DMA,
                   pltpu.SemaphoreType.DMA,
                   pltpu.SemaphoreType.DMA,
                   pltpu.SemaphoreType.DMA,
                   pltpu.SemaphoreType.DMA,
                   pltpu.SemaphoreType.DMA],
)
def _sc_route(bins_hbm, hist_hbm, x_hbm, xl_hbm,
              xs_hbm, xls_hbm, dst_hbm, head_hbm,
              hist_v, bins_v, dst_v, xl_v, xrow_v, head_v,
              sg0, sg1, sg2, sg3, ss0, ss1, ss2, ss3):
    wid = lax.axis_index("s") * _NC + lax.axis_index("c")
    base = wid * _R
    lane = lax.iota(jnp.int32, 16)
    zeros16 = jnp.zeros((16,), jnp.int32)
    sem_g = (sg0, sg1, sg2, sg3)
    sem_s = (ss0, ss1, ss2, ss3)

    pltpu.sync_copy(hist_hbm, hist_v)
    pltpu.sync_copy(bins_hbm.at[pl.ds(base, _R)], bins_v)
    pltpu.sync_copy(xl_hbm.at[pl.ds(base, _R)], xl_v)

    def acc(wp, carry):
        tot, mycum = carry
        row = hist_v[pl.ds(wp * 16, 16)]
        return tot + row, mycum + jnp.where(wp < wid, row, 0)

    tot, mycum = lax.fori_loop(0, _NW, acc, (zeros16, zeros16))

    rt = ((tot + (_T - 1)) // _T) * _T
    pexc = jnp.cumsum(rt) - rt
    off = pexc + mycum

    offs = [jnp.sum(jnp.where(lane == h, off, 0)) for h in range(_H)]
    pstarts = [jnp.sum(jnp.where(lane == h, pexc, 0)) // _T
               for h in range(1, _H)]

    @pl.when(wid == 0)
    def _():
        for v in range(_NHPAD // 16):
            tiles = lane + v * 16
            hv = zeros16
            for ps in pstarts:
                hv = hv + (tiles >= ps).astype(jnp.int32)
            head_v[pl.ds(v * 16, 16)] = hv
        pltpu.sync_copy(head_v, head_hbm)

    def rank_body(g, carrys):
        bv = bins_v[pl.ds(g * 16, 16)]
        dst = zeros16
        new = []
        for h in range(_H):
            m = (bv == h).astype(jnp.int32)
            cs = jnp.cumsum(m)
            dst = dst + m * (offs[h] + carrys[h] + cs - 1)
            new.append(carrys[h] + jnp.sum(m))
        dst_v[pl.ds(g * 16, 16)] = dst
        return tuple(new)

    lax.fori_loop(0, _NG, rank_body, (jnp.int32(0),) * _H)
    pltpu.sync_copy(dst_v, dst_hbm.at[pl.ds(base, _R)])

    def start_gather(b, g):
        pltpu.async_copy(
            x_hbm.at[pl.ds(base + g * 16, 16), pl.ds(0, _D)],
            xrow_v.at[b], sem_g[b])

    for b in range(_NBUF):
        start_gather(b, b)

    @pl.loop(0, _NG, step=_NBUF)
    def move(go):
        for b in range(_NBUF):
            g = go + b
            pltpu.make_async_copy(
                x_hbm.at[pl.ds(0, 16), pl.ds(0, _D)],
                xrow_v.at[b], sem_g[b]).wait()
            dg = dst_v[pl.ds(g * 16, 16)]
            c1 = pltpu.async_copy(xrow_v.at[b], xs_hbm.at[dg], sem_s[b])
            c2 = pltpu.async_copy(xl_v.at[pl.ds(g * 16, 16)],
                                  xls_hbm.at[dg], sem_s[b])
            c1.wait()
            c2.wait()

            @pl.when(g + _NBUF < _NG)
            def _():
                start_gather(b, g + _NBUF)


@functools.partial(
    pl.kernel, mesh=_mesh, compiler_params=_sc_params,
    out_type=jax.ShapeDtypeStruct((_B, _DO), jnp.float32),
    scratch_types=[pltpu.VMEM((_R,), jnp.int32),
                   pltpu.VMEM((_NBUF, 16, _DO), jnp.float32),
                   pltpu.SemaphoreType.DMA,
                   pltpu.SemaphoreType.DMA,
                   pltpu.SemaphoreType.DMA,
                   pltpu.SemaphoreType.DMA],
)
def _sc_unsort(dst_hbm, ys_hbm, out_hbm, dst_v, row_v, sg0, sg1, sg2, sg3):
    wid = lax.axis_index("s") * _NC + lax.axis_index("c")
    base = wid * _R
    sem_g = (sg0, sg1, sg2, sg3)
    pltpu.sync_copy(dst_hbm.at[pl.ds(base, _R)], dst_v)

    def start_gather(b, g):
        dg = dst_v[pl.ds(g * 16, 16)]
        pltpu.async_copy(ys_hbm.at[dg], row_v.at[b], sem_g[b])

    for b in range(_NBUF):
        start_gather(b, b)

    @pl.loop(0, _NG, step=_NBUF)
    def move(go):
        for b in range(_NBUF):
            g = go + b
            pltpu.make_async_copy(
                ys_hbm.at[pl.ds(0, 16)], row_v.at[b], sem_g[b]).wait()
            pltpu.sync_copy(row_v.at[b],
                            out_hbm.at[pl.ds(base + g * 16, 16)])

            @pl.when(g + _NBUF < _NG)
            def _():
                start_gather(b, g + _NBUF)


def _tc_body(head_ref, xs_ref, xl_ref, w0_ref, w0l_ref, b0_ref,
             w1_ref, b1_ref, t1_ref, w2_ref, b2_ref, t2_ref, out_ref):
    x = xs_ref[:]
    t = xs_ref[:, 0:1]
    xl = xl_ref[:]
    h = jnp.maximum(
        jnp.dot(x.astype(jnp.bfloat16), w0_ref[0],
                preferred_element_type=jnp.float32)
        + xl * w0l_ref[0, 0] + b0_ref[0, 0], 0.0)
    h = jnp.maximum(
        jnp.dot(h.astype(jnp.bfloat16), w1_ref[0],
                preferred_element_type=jnp.float32)
        + t * t1_ref[0, 0] + b1_ref[0, 0], 0.0)
    out_ref[:] = jnp.maximum(
        jnp.dot(h.astype(jnp.bfloat16), w2_ref[0],
                preferred_element_type=jnp.float32)
        + t * t2_ref[0, 0] + b2_ref[0, 0], 0.0)


def _tc_mlp(head, xs, xl, W0c, w0l, b0, W1, b1, tw1, W2, b2, tw2):
    grid_spec = pltpu.PrefetchScalarGridSpec(
        num_scalar_prefetch=1,
        grid=(_NT,),
        in_specs=[
            pl.BlockSpec((_T, _D), lambda i, hd: (i, 0)),
            pl.BlockSpec((_T, 1), lambda i, hd: (i, 0)),
            pl.BlockSpec((1, _D, _D), lambda i, hd: (hd[i], 0, 0)),
            pl.BlockSpec((1, 1, _D), lambda i, hd: (hd[i], 0, 0)),
            pl.BlockSpec((1, 1, _D), lambda i, hd: (hd[i], 0, 0)),
            pl.BlockSpec((1, _D, _D), lambda i, hd: (hd[i], 0, 0)),
            pl.BlockSpec((1, 1, _D), lambda i, hd: (hd[i], 0, 0)),
            pl.BlockSpec((1, 1, _D), lambda i, hd: (hd[i], 0, 0)),
            pl.BlockSpec((1, _D, _DO), lambda i, hd: (hd[i], 0, 0)),
            pl.BlockSpec((1, 1, _DO), lambda i, hd: (hd[i], 0, 0)),
            pl.BlockSpec((1, 1, _DO), lambda i, hd: (hd[i], 0, 0)),
        ],
        out_specs=pl.BlockSpec((_T, _DO), lambda i, hd: (i, 0)),
    )
    return pl.pallas_call(
        _tc_body,
        grid_spec=grid_spec,
        out_shape=jax.ShapeDtypeStruct((_NTOT, _DO), jnp.float32),
        compiler_params=pltpu.CompilerParams(
            dimension_semantics=("arbitrary",),
        ),
    )(head, xs, xl, W0c, w0l, b0, W1, b1, tw1, W2, b2, tw2)


def kernel(x, W0, b0, tw0, W1, b1, tw1, W2, b2, tw2):
    t_all = x[:, 0]
    xlast = x[:, _D]
    W0c = jnp.concatenate([tw0, W0[:, : _D - 1, :]], axis=1).astype(jnp.bfloat16)
    W1 = W1.astype(jnp.bfloat16)
    W2 = W2.astype(jnp.bfloat16)
    w0l = W0[:, _D - 1 : _D, :]
    b0r = b0.reshape(_H, 1, _D)
    b1r = b1.reshape(_H, 1, _D)
    b2r = b2.reshape(_H, 1, _DO)

    bins, hist = _sc_bin_hist(t_all)
    xs, xls, dst, head = _sc_route(bins, hist, x, xlast)
    ys = _tc_mlp(head, xs, xls.reshape(_NTOT, 1),
                 W0c, w0l, b0r, W1, b1r, tw1, W2, b2r, tw2)
    return _sc_unsort(dst, ys)


# speedup vs baseline: 1.0275x; 1.0275x over previous
import functools

import jax
import jax.numpy as jnp
from jax import lax
from jax.experimental import pallas as pl
from jax.experimental.pallas import tpu as pltpu
from jax.experimental.pallas import tpu_sc as plsc

_PT = (0.6, 0.7, 0.8, 0.9)
_B = 16384
_H = 5
_D = 1024
_DO = 512
_T = 256
_NTOT = _B + _H * _T
_NT = _NTOT // _T
_NHPAD = 80
_NC = 2
_NS = 16
_NW = _NC * _NS
_R = _B // _NW
_NG = _R // 16
_NBUF = 4

_mesh = plsc.VectorSubcoreMesh(core_axis_name="c", subcore_axis_name="s")
_sc_params = pltpu.CompilerParams(needs_layout_passes=False)


@functools.partial(
    pl.kernel, mesh=_mesh, compiler_params=_sc_params,
    out_type=[jax.ShapeDtypeStruct((_B,), jnp.int32),
              jax.ShapeDtypeStruct((_NW * 16,), jnp.int32)],
    scratch_types=[pltpu.VMEM((_R,), jnp.float32),
                   pltpu.VMEM((_R,), jnp.int32),
                   pltpu.VMEM((16,), jnp.int32)],
)
def _sc_bin_hist(t_hbm, bins_hbm, hist_hbm, t_v, bins_v, hist_v):
    wid = lax.axis_index("s") * _NC + lax.axis_index("c")
    base = wid * _R
    lane = lax.iota(jnp.int32, 16)
    pltpu.sync_copy(t_hbm.at[pl.ds(base, _R)], t_v)

    def body(g, hist):
        tv = t_v[pl.ds(g * 16, 16)]
        bv = sum((tv >= p).astype(jnp.int32) for p in _PT)
        bins_v[pl.ds(g * 16, 16)] = bv
        for h in range(_H):
            cnt = jnp.sum((bv == h).astype(jnp.int32))
            hist = hist + jnp.where(lane == h, cnt, 0)
        return hist

    hist = lax.fori_loop(0, _NG, body, jnp.zeros((16,), jnp.int32))
    hist_v[...] = hist
    pltpu.sync_copy(bins_v, bins_hbm.at[pl.ds(base, _R)])
    pltpu.sync_copy(hist_v, hist_hbm.at[pl.ds(wid * 16, 16)])


@functools.partial(
    pl.kernel, mesh=_mesh, compiler_params=_sc_params,
    out_type=[jax.ShapeDtypeStruct((_NTOT, _D), jnp.float32),
              jax.ShapeDtypeStruct((_NTOT,), jnp.float32),
              jax.ShapeDtypeStruct((_B,), jnp.int32),
              jax.ShapeDtypeStruct((_NHPAD,), jnp.int32)],
    scratch_types=[pltpu.VMEM((_NW * 16,), jnp.int32),
                   pltpu.VMEM((_R,), jnp.int32),
                   pltpu.VMEM((_R,), jnp.int32),
                   pltpu.VMEM((_R,), jnp.float32),
                   pltpu.VMEM((_NBUF, 16, _D), jnp.float32),
                   pltpu.VMEM((_NHPAD,), jnp.int32),
                   pltpu.SemaphoreType.DMA,
                   pltpu.SemaphoreType.DMA,
                   pltpu.SemaphoreType.DMA,
                   pltpu.SemaphoreType.DMA,
                   pltpu.SemaphoreType.DMA,
                   pltpu.SemaphoreType.DMA,
                   pltpu.SemaphoreType.DMA,
                   pltpu.SemaphoreType.DMA],
)
def _sc_route(bins_hbm, hist_hbm, x_hbm, xl_hbm,
              xs_hbm, xls_hbm, dst_hbm, head_hbm,
              hist_v, bins_v, dst_v, xl_v, xrow_v, head_v,
              sg0, sg1, sg2, sg3, ss0, ss1, ss2, ss3):
    wid = lax.axis_index("s") * _NC + lax.axis_index("c")
    base = wid * _R
    lane = lax.iota(jnp.int32, 16)
    zeros16 = jnp.zeros((16,), jnp.int32)
    sem_g = (sg0, sg1, sg2, sg3)
    sem_s = (ss0, ss1, ss2, ss3)

    pltpu.sync_copy(hist_hbm, hist_v)
    pltpu.sync_copy(bins_hbm.at[pl.ds(base, _R)], bins_v)
    pltpu.sync_copy(xl_hbm.at[pl.ds(base, _R)], xl_v)

    def acc(wp, carry):
        tot, mycum = carry
        row = hist_v[pl.ds(wp * 16, 16)]
        return tot + row, mycum + jnp.where(wp < wid, row, 0)

    tot, mycum = lax.fori_loop(0, _NW, acc, (zeros16, zeros16))

    rt = ((tot + (_T - 1)) // _T) * _T
    pexc = jnp.cumsum(rt) - rt
    off = pexc + mycum

    offs = [jnp.sum(jnp.where(lane == h, off, 0)) for h in range(_H)]
    pstarts = [jnp.sum(jnp.where(lane == h, pexc, 0)) // _T
               for h in range(1, _H)]

    @pl.when(wid == 0)
    def _():
        for v in range(_NHPAD // 16):
            tiles = lane + v * 16
            hv = zeros16
            for ps in pstarts:
                hv = hv + (tiles >= ps).astype(jnp.int32)
            head_v[pl.ds(v * 16, 16)] = hv
        pltpu.sync_copy(head_v, head_hbm)

    def rank_body(g, carrys):
        bv = bins_v[pl.ds(g * 16, 16)]
        dst = zeros16
        new = []
        for h in range(_H):
            m = (bv == h).astype(jnp.int32)
            cs = jnp.cumsum(m)
            dst = dst + m * (offs[h] + carrys[h] + cs - 1)
            new.append(carrys[h] + jnp.sum(m))
        dst_v[pl.ds(g * 16, 16)] = dst
        return tuple(new)

    lax.fori_loop(0, _NG, rank_body, (jnp.int32(0),) * _H)
    pltpu.sync_copy(dst_v, dst_hbm.at[pl.ds(base, _R)])

    def start_gather(b, g):
        pltpu.async_copy(
            x_hbm.at[pl.ds(base + g * 16, 16), pl.ds(0, _D)],
            xrow_v.at[b], sem_g[b])

    for b in range(_NBUF):
        start_gather(b, b)

    def wait_scatters(b):
        pltpu.make_async_copy(xrow_v.at[b], xs_hbm.at[pl.ds(0, 16)],
                              sem_s[b]).wait()
        pltpu.make_async_copy(xl_v.at[pl.ds(0, 16)],
                              xls_hbm.at[pl.ds(0, 16)], sem_s[b]).wait()

    @pl.loop(0, _NG, step=_NBUF)
    def move(go):
        for b in range(_NBUF):
            g = go + b
            pltpu.make_async_copy(
                x_hbm.at[pl.ds(0, 16), pl.ds(0, _D)],
                xrow_v.at[b], sem_g[b]).wait()
            dg = dst_v[pl.ds(g * 16, 16)]
            pltpu.async_copy(xrow_v.at[b], xs_hbm.at[dg], sem_s[b])
            pltpu.async_copy(xl_v.at[pl.ds(g * 16, 16)],
                             xls_hbm.at[dg], sem_s[b])
            gpre = g + _NBUF - 2
            bpre = (b + _NBUF - 2) % _NBUF

            @pl.when(jnp.logical_and(gpre >= _NBUF, gpre < _NG))
            def _():
                wait_scatters(bpre)
                start_gather(bpre, gpre)

    for b in range(_NBUF):
        wait_scatters(b)


@functools.partial(
    pl.kernel, mesh=_mesh, compiler_params=_sc_params,
    out_type=jax.ShapeDtypeStruct((_B, _DO), jnp.float32),
    scratch_types=[pltpu.VMEM((_R,), jnp.int32),
                   pltpu.VMEM((_NBUF, 16, _DO), jnp.float32),
                   pltpu.SemaphoreType.DMA,
                   pltpu.SemaphoreType.DMA,
                   pltpu.SemaphoreType.DMA,
                   pltpu.SemaphoreType.DMA,
                   pltpu.SemaphoreType.DMA,
                   pltpu.SemaphoreType.DMA,
                   pltpu.SemaphoreType.DMA,
                   pltpu.SemaphoreType.DMA],
)
def _sc_unsort(dst_hbm, ys_hbm, out_hbm, dst_v, row_v,
               sg0, sg1, sg2, sg3, so0, so1, so2, so3):
    wid = lax.axis_index("s") * _NC + lax.axis_index("c")
    base = wid * _R
    sem_g = (sg0, sg1, sg2, sg3)
    sem_o = (so0, so1, so2, so3)
    pltpu.sync_copy(dst_hbm.at[pl.ds(base, _R)], dst_v)

    def start_gather(b, g):
        dg = dst_v[pl.ds(g * 16, 16)]
        pltpu.async_copy(ys_hbm.at[dg], row_v.at[b], sem_g[b])

    def wait_out(b):
        pltpu.make_async_copy(row_v.at[b], out_hbm.at[pl.ds(0, 16)],
                              sem_o[b]).wait()

    for b in range(_NBUF):
        start_gather(b, b)

    @pl.loop(0, _NG, step=_NBUF)
    def move(go):
        for b in range(_NBUF):
            g = go + b
            pltpu.make_async_copy(
                ys_hbm.at[pl.ds(0, 16)], row_v.at[b], sem_g[b]).wait()
            pltpu.async_copy(row_v.at[b],
                             out_hbm.at[pl.ds(base + g * 16, 16)], sem_o[b])
            gpre = g + _NBUF - 2
            bpre = (b + _NBUF - 2) % _NBUF

            @pl.when(jnp.logical_and(gpre >= _NBUF, gpre < _NG))
            def _():
                wait_out(bpre)
                start_gather(bpre, gpre)

    for b in range(_NBUF):
        wait_out(b)


def _tc_body(head_ref, xs_ref, xl_ref, w0_ref, w0l_ref, b0_ref,
             w1_ref, b1_ref, t1_ref, w2_ref, b2_ref, t2_ref, out_ref):
    x = xs_ref[:]
    t = xs_ref[:, 0:1]
    xl = xl_ref[:]
    h = jnp.maximum(
        jnp.dot(x, w0_ref[0], preferred_element_type=jnp.float32)
        + xl * w0l_ref[0, 0] + b0_ref[0, 0], 0.0)
    h = jnp.maximum(
        jnp.dot(h, w1_ref[0], preferred_element_type=jnp.float32)
        + t * t1_ref[0, 0] + b1_ref[0, 0], 0.0)
    out_ref[:] = jnp.maximum(
        jnp.dot(h, w2_ref[0], preferred_element_type=jnp.float32)
        + t * t2_ref[0, 0] + b2_ref[0, 0], 0.0)


def _tc_mlp(head, xs, xl, W0c, w0l, b0, W1, b1, tw1, W2, b2, tw2):
    grid_spec = pltpu.PrefetchScalarGridSpec(
        num_scalar_prefetch=1,
        grid=(_NT,),
        in_specs=[
            pl.BlockSpec((_T, _D), lambda i, hd: (i, 0)),
            pl.BlockSpec((_T, 1), lambda i, hd: (i, 0)),
            pl.BlockSpec((1, _D, _D), lambda i, hd: (hd[i], 0, 0)),
            pl.BlockSpec((1, 1, _D), lambda i, hd: (hd[i], 0, 0)),
            pl.BlockSpec((1, 1, _D), lambda i, hd: (hd[i], 0, 0)),
            pl.BlockSpec((1, _D, _D), lambda i, hd: (hd[i], 0, 0)),
            pl.BlockSpec((1, 1, _D), lambda i, hd: (hd[i], 0, 0)),
            pl.BlockSpec((1, 1, _D), lambda i, hd: (hd[i], 0, 0)),
            pl.BlockSpec((1, _D, _DO), lambda i, hd: (hd[i], 0, 0)),
            pl.BlockSpec((1, 1, _DO), lambda i, hd: (hd[i], 0, 0)),
            pl.BlockSpec((1, 1, _DO), lambda i, hd: (hd[i], 0, 0)),
        ],
        out_specs=pl.BlockSpec((_T, _DO), lambda i, hd: (i, 0)),
    )
    return pl.pallas_call(
        _tc_body,
        grid_spec=grid_spec,
        out_shape=jax.ShapeDtypeStruct((_NTOT, _DO), jnp.float32),
        compiler_params=pltpu.CompilerParams(
            dimension_semantics=("arbitrary",),
        ),
    )(head, xs, xl, W0c, w0l, b0, W1, b1, tw1, W2, b2, tw2)


def kernel(x, W0, b0, tw0, W1, b1, tw1, W2, b2, tw2):
    t_all = x[:, 0]
    xlast = x[:, _D]
    W0c = jnp.concatenate([tw0, W0[:, : _D - 1, :]], axis=1)
    w0l = W0[:, _D - 1 : _D, :]
    b0r = b0.reshape(_H, 1, _D)
    b1r = b1.reshape(_H, 1, _D)
    b2r = b2.reshape(_H, 1, _DO)

    bins, hist = _sc_bin_hist(t_all)
    xs, xls, dst, head = _sc_route(bins, hist, x, xlast)
    ys = _tc_mlp(head, xs, xls.reshape(_NTOT, 1),
                 W0c, w0l, b0r, W1, b1r, tw1, W2, b2r, tw2)
    return _sc_unsort(dst, ys)


# speedup vs baseline: 1.0392x; 1.0115x over previous
import functools

import jax
import jax.numpy as jnp
from jax import lax
from jax.experimental import pallas as pl
from jax.experimental.pallas import tpu as pltpu
from jax.experimental.pallas import tpu_sc as plsc

_PT = (0.6, 0.7, 0.8, 0.9)
_B = 16384
_H = 5
_D = 1024
_DW = 1152
_DO = 512
_T = 256
_NTOT = _B + _H * _T
_NT = _NTOT // _T
_NHPAD = 80
_NC = 2
_NS = 16
_NW = _NC * _NS
_R = _B // _NW
_NG = _R // 16
_NBUF = 4

_mesh = plsc.VectorSubcoreMesh(core_axis_name="c", subcore_axis_name="s")
_sc_params = pltpu.CompilerParams(needs_layout_passes=False)


@functools.partial(
    pl.kernel, mesh=_mesh, compiler_params=_sc_params,
    out_type=[jax.ShapeDtypeStruct((_B,), jnp.int32),
              jax.ShapeDtypeStruct((_NW * 16,), jnp.int32)],
    scratch_types=[pltpu.VMEM((_R,), jnp.float32),
                   pltpu.VMEM((_R,), jnp.int32),
                   pltpu.VMEM((16,), jnp.int32)],
)
def _sc_bin_hist(t_hbm, bins_hbm, hist_hbm, t_v, bins_v, hist_v):
    wid = lax.axis_index("s") * _NC + lax.axis_index("c")
    base = wid * _R
    lane = lax.iota(jnp.int32, 16)
    pltpu.sync_copy(t_hbm.at[pl.ds(base, _R)], t_v)

    def body(g, hist):
        tv = t_v[pl.ds(g * 16, 16)]
        bv = sum((tv >= p).astype(jnp.int32) for p in _PT)
        bins_v[pl.ds(g * 16, 16)] = bv
        for h in range(_H):
            cnt = jnp.sum((bv == h).astype(jnp.int32))
            hist = hist + jnp.where(lane == h, cnt, 0)
        return hist

    hist = lax.fori_loop(0, _NG, body, jnp.zeros((16,), jnp.int32))
    hist_v[...] = hist
    pltpu.sync_copy(bins_v, bins_hbm.at[pl.ds(base, _R)])
    pltpu.sync_copy(hist_v, hist_hbm.at[pl.ds(wid * 16, 16)])


@functools.partial(
    pl.kernel, mesh=_mesh, compiler_params=_sc_params,
    out_type=[jax.ShapeDtypeStruct((_NTOT, _DW), jnp.float32),
              jax.ShapeDtypeStruct((_B,), jnp.int32),
              jax.ShapeDtypeStruct((_NHPAD,), jnp.int32)],
    scratch_types=[pltpu.VMEM((_NW * 16,), jnp.int32),
                   pltpu.VMEM((_R,), jnp.int32),
                   pltpu.VMEM((_R,), jnp.int32),
                   pltpu.VMEM((_R,), jnp.float32),
                   pltpu.VMEM((_NBUF, 16, _DW), jnp.float32),
                   pltpu.VMEM((_NHPAD,), jnp.int32),
                   pltpu.SemaphoreType.DMA,
                   pltpu.SemaphoreType.DMA,
                   pltpu.SemaphoreType.DMA,
                   pltpu.SemaphoreType.DMA,
                   pltpu.SemaphoreType.DMA,
                   pltpu.SemaphoreType.DMA,
                   pltpu.SemaphoreType.DMA,
                   pltpu.SemaphoreType.DMA],
)
def _sc_route(bins_hbm, hist_hbm, x_hbm, xl_hbm,
              xs_hbm, dst_hbm, head_hbm,
              hist_v, bins_v, dst_v, xl_v, xrow_v, head_v,
              sg0, sg1, sg2, sg3, ss0, ss1, ss2, ss3):
    wid = lax.axis_index("s") * _NC + lax.axis_index("c")
    base = wid * _R
    lane = lax.iota(jnp.int32, 16)
    zeros16 = jnp.zeros((16,), jnp.int32)
    zerosf16 = jnp.zeros((16,), jnp.float32)
    sem_g = (sg0, sg1, sg2, sg3)
    sem_s = (ss0, ss1, ss2, ss3)

    def zinit(r, carry):
        for b in range(_NBUF):
            for q in range((_DW - _D) // 16):
                xrow_v[b, r, pl.ds(_D + q * 16, 16)] = zerosf16
        return carry

    lax.fori_loop(0, 16, zinit, jnp.int32(0))

    pltpu.sync_copy(hist_hbm, hist_v)
    pltpu.sync_copy(bins_hbm.at[pl.ds(base, _R)], bins_v)
    pltpu.sync_copy(xl_hbm.at[pl.ds(base, _R)], xl_v)

    def acc(wp, carry):
        tot, mycum = carry
        row = hist_v[pl.ds(wp * 16, 16)]
        return tot + row, mycum + jnp.where(wp < wid, row, 0)

    tot, mycum = lax.fori_loop(0, _NW, acc, (zeros16, zeros16))

    rt = ((tot + (_T - 1)) // _T) * _T
    pexc = jnp.cumsum(rt) - rt
    off = pexc + mycum

    offs = [jnp.sum(jnp.where(lane == h, off, 0)) for h in range(_H)]
    pstarts = [jnp.sum(jnp.where(lane == h, pexc, 0)) // _T
               for h in range(1, _H)]

    @pl.when(wid == 0)
    def _():
        for v in range(_NHPAD // 16):
            tiles = lane + v * 16
            hv = zeros16
            for ps in pstarts:
                hv = hv + (tiles >= ps).astype(jnp.int32)
            head_v[pl.ds(v * 16, 16)] = hv
        pltpu.sync_copy(head_v, head_hbm)

    def rank_body(g, carrys):
        bv = bins_v[pl.ds(g * 16, 16)]
        dst = zeros16
        new = []
        for h in range(_H):
            m = (bv == h).astype(jnp.int32)
            cs = jnp.cumsum(m)
            dst = dst + m * (offs[h] + carrys[h] + cs - 1)
            new.append(carrys[h] + jnp.sum(m))
        dst_v[pl.ds(g * 16, 16)] = dst
        return tuple(new)

    lax.fori_loop(0, _NG, rank_body, (jnp.int32(0),) * _H)
    pltpu.sync_copy(dst_v, dst_hbm.at[pl.ds(base, _R)])

    def start_gather(b, g):
        pltpu.async_copy(
            x_hbm.at[pl.ds(base + g * 16, 16), pl.ds(0, _D)],
            xrow_v.at[b, :, pl.ds(0, _D)], sem_g[b])

    for b in range(_NBUF):
        start_gather(b, b)

    def wait_scatter(b):
        pltpu.make_async_copy(xrow_v.at[b], xs_hbm.at[pl.ds(0, 16)],
                              sem_s[b]).wait()

    @pl.loop(0, _NG, step=_NBUF)
    def move(go):
        for b in range(_NBUF):
            g = go + b
            pltpu.make_async_copy(
                x_hbm.at[pl.ds(0, 16), pl.ds(0, _D)],
                xrow_v.at[b, :, pl.ds(0, _D)], sem_g[b]).wait()
            xlg = xl_v[pl.ds(g * 16, 16)]
            plsc.store_scatter(
                xrow_v,
                [jnp.full((16,), b, jnp.int32), lane,
                 jnp.full((16,), _D, jnp.int32)], xlg)
            dg = dst_v[pl.ds(g * 16, 16)]
            pltpu.async_copy(xrow_v.at[b], xs_hbm.at[dg], sem_s[b])
            gpre = g + _NBUF - 2
            bpre = (b + _NBUF - 2) % _NBUF

            @pl.when(jnp.logical_and(gpre >= _NBUF, gpre < _NG))
            def _():
                wait_scatter(bpre)
                start_gather(bpre, gpre)

    for b in range(_NBUF):
        wait_scatter(b)


@functools.partial(
    pl.kernel, mesh=_mesh, compiler_params=_sc_params,
    out_type=jax.ShapeDtypeStruct((_B, _DO), jnp.float32),
    scratch_types=[pltpu.VMEM((_R,), jnp.int32),
                   pltpu.VMEM((_NBUF, 16, _DO), jnp.float32),
                   pltpu.SemaphoreType.DMA,
                   pltpu.SemaphoreType.DMA,
                   pltpu.SemaphoreType.DMA,
                   pltpu.SemaphoreType.DMA,
                   pltpu.SemaphoreType.DMA,
                   pltpu.SemaphoreType.DMA,
                   pltpu.SemaphoreType.DMA,
                   pltpu.SemaphoreType.DMA],
)
def _sc_unsort(dst_hbm, ys_hbm, out_hbm, dst_v, row_v,
               sg0, sg1, sg2, sg3, so0, so1, so2, so3):
    wid = lax.axis_index("s") * _NC + lax.axis_index("c")
    base = wid * _R
    sem_g = (sg0, sg1, sg2, sg3)
    sem_o = (so0, so1, so2, so3)
    pltpu.sync_copy(dst_hbm.at[pl.ds(base, _R)], dst_v)

    def start_gather(b, g):
        dg = dst_v[pl.ds(g * 16, 16)]
        pltpu.async_copy(ys_hbm.at[dg], row_v.at[b], sem_g[b])

    def wait_out(b):
        pltpu.make_async_copy(row_v.at[b], out_hbm.at[pl.ds(0, 16)],
                              sem_o[b]).wait()

    for b in range(_NBUF):
        start_gather(b, b)

    @pl.loop(0, _NG, step=_NBUF)
    def move(go):
        for b in range(_NBUF):
            g = go + b
            pltpu.make_async_copy(
                ys_hbm.at[pl.ds(0, 16)], row_v.at[b], sem_g[b]).wait()
            pltpu.async_copy(row_v.at[b],
                             out_hbm.at[pl.ds(base + g * 16, 16)], sem_o[b])
            gpre = g + _NBUF - 2
            bpre = (b + _NBUF - 2) % _NBUF

            @pl.when(jnp.logical_and(gpre >= _NBUF, gpre < _NG))
            def _():
                wait_out(bpre)
                start_gather(bpre, gpre)

    for b in range(_NBUF):
        wait_out(b)


def _tc_body(head_ref, xs_ref, w0_ref, b0_ref,
             w1_ref, b1_ref, t1_ref, w2_ref, b2_ref, t2_ref, out_ref):
    x = xs_ref[:]
    t = xs_ref[:, 0:1]
    h = jnp.maximum(
        jnp.dot(x, w0_ref[0], preferred_element_type=jnp.float32)
        + b0_ref[0, 0], 0.0)
    h = jnp.maximum(
        jnp.dot(h, w1_ref[0], preferred_element_type=jnp.float32)
        + t * t1_ref[0, 0] + b1_ref[0, 0], 0.0)
    out_ref[:] = jnp.maximum(
        jnp.dot(h, w2_ref[0], preferred_element_type=jnp.float32)
        + t * t2_ref[0, 0] + b2_ref[0, 0], 0.0)


def _tc_mlp(head, xs, W0f, b0, W1, b1, tw1, W2, b2, tw2):
    grid_spec = pltpu.PrefetchScalarGridSpec(
        num_scalar_prefetch=1,
        grid=(_NT,),
        in_specs=[
            pl.BlockSpec((_T, _DW), lambda i, hd: (i, 0)),
            pl.BlockSpec((1, _DW, _D), lambda i, hd: (hd[i], 0, 0)),
            pl.BlockSpec((1, 1, _D), lambda i, hd: (hd[i], 0, 0)),
            pl.BlockSpec((1, _D, _D), lambda i, hd: (hd[i], 0, 0)),
            pl.BlockSpec((1, 1, _D), lambda i, hd: (hd[i], 0, 0)),
            pl.BlockSpec((1, 1, _D), lambda i, hd: (hd[i], 0, 0)),
            pl.BlockSpec((1, _D, _DO), lambda i, hd: (hd[i], 0, 0)),
            pl.BlockSpec((1, 1, _DO), lambda i, hd: (hd[i], 0, 0)),
            pl.BlockSpec((1, 1, _DO), lambda i, hd: (hd[i], 0, 0)),
        ],
        out_specs=pl.BlockSpec((_T, _DO), lambda i, hd: (i, 0)),
    )
    return pl.pallas_call(
        _tc_body,
        grid_spec=grid_spec,
        out_shape=jax.ShapeDtypeStruct((_NTOT, _DO), jnp.float32),
        compiler_params=pltpu.CompilerParams(
            dimension_semantics=("arbitrary",),
        ),
    )(head, xs, W0f, b0, W1, b1, tw1, W2, b2, tw2)


def kernel(x, W0, b0, tw0, W1, b1, tw1, W2, b2, tw2):
    t_all = x[:, 0]
    xlast = x[:, _D]
    W0f = jnp.concatenate(
        [tw0, W0, jnp.zeros((_H, _DW - _D - 1, _D), W0.dtype)], axis=1)
    b0r = b0.reshape(_H, 1, _D)
    b1r = b1.reshape(_H, 1, _D)
    b2r = b2.reshape(_H, 1, _DO)

    bins, hist = _sc_bin_hist(t_all)
    xs, dst, head = _sc_route(bins, hist, x, xlast)
    ys = _tc_mlp(head, xs, W0f, b0r, W1, b1r, tw1, W2, b2r, tw2)
    return _sc_unsort(dst, ys)
